# baseline (speedup 1.0000x reference)
import jax
import jax.numpy as jnp
from jax.experimental import pallas as pl
from jax.experimental.pallas import tpu as pltpu

BUCKET_W = 4.0
N_ROWS = 1_000_000
D = 64
BC = 32768
GRID_A = -(-N_ROWS // BC)
BS = 1048576
GRID_B = -(-N_ROWS // BS)
HBUF = GRID_B * BS


def _body(x_ref, a_ref, b_ref, o_ref, hbuf, min_sc):
    i = pl.program_id(0)

    @pl.when(i < GRID_A)
    def _():
        proj = jax.lax.dot_general(
            a_ref[...], x_ref[...],
            dimension_numbers=(((1,), (0,)), ((), ())),
            preferred_element_type=jnp.float32,
        )
        h = jnp.floor((proj + b_ref[0]) * (1.0 / BUCKET_W))

        @pl.when(i == 0)
        def _():
            min_sc[0] = jnp.min(h)

        @pl.when((i > 0) & (i < GRID_A - 1))
        def _():
            min_sc[0] = jnp.minimum(min_sc[0], jnp.min(h))

        @pl.when(i == GRID_A - 1)
        def _():
            cols = i * BC + jax.lax.broadcasted_iota(jnp.int32, (1, BC), 1)
            hm = jnp.min(jnp.where(cols < N_ROWS, h, jnp.inf))
            min_sc[0] = jnp.minimum(min_sc[0], hm)

        hbuf[pl.ds(i * BC, BC)] = h.reshape(BC).astype(jnp.int32)

    @pl.when(i >= GRID_A)
    def _():
        j = i - GRID_A
        m = min_sc[0].astype(jnp.int32)
        o_ref[...] = hbuf[pl.ds(j * BS, BS)] - m


def kernel(inputs, a, b):
    xt = inputs.T
    a2 = a.reshape(1, D)
    out = pl.pallas_call(
        _body,
        grid=(GRID_A + GRID_B,),
        in_specs=[
            pl.BlockSpec((D, BC), lambda i: (0, jnp.minimum(i, GRID_A - 1))),
            pl.BlockSpec((1, D), lambda i: (0, 0)),
            pl.BlockSpec(memory_space=pltpu.SMEM),
        ],
        out_specs=pl.BlockSpec((BS,), lambda i: (jnp.maximum(i - GRID_A, 0),)),
        out_shape=jax.ShapeDtypeStruct((N_ROWS,), jnp.int32),
        scratch_shapes=[
            pltpu.VMEM((HBUF,), jnp.int32),
            pltpu.SMEM((1,), jnp.float32),
        ],
    )(xt, a2, b)
    return out


# speedup vs baseline: 1.0007x; 1.0007x over previous
import jax
import jax.numpy as jnp
from jax.experimental import pallas as pl
from jax.experimental.pallas import tpu as pltpu

BUCKET_W = 4.0
N_ROWS = 1_000_000
D = 64
BC = 40960
GRID_A = -(-N_ROWS // BC)
BS = 524288
GRID_B = -(-N_ROWS // BS)
HBUF = GRID_B * BS


def _body(x_ref, a_ref, b_ref, o_ref, hbuf, min_sc):
    i = pl.program_id(0)

    @pl.when(i < GRID_A)
    def _():
        proj = jax.lax.dot_general(
            a_ref[...], x_ref[...],
            dimension_numbers=(((1,), (0,)), ((), ())),
            preferred_element_type=jnp.float32,
        )
        h = jnp.floor((proj + b_ref[0]) * (1.0 / BUCKET_W))

        @pl.when(i == 0)
        def _():
            min_sc[0] = jnp.min(h)

        @pl.when((i > 0) & (i < GRID_A - 1))
        def _():
            min_sc[0] = jnp.minimum(min_sc[0], jnp.min(h))

        @pl.when(i == GRID_A - 1)
        def _():
            cols = i * BC + jax.lax.broadcasted_iota(jnp.int32, (1, BC), 1)
            hm = jnp.min(jnp.where(cols < N_ROWS, h, jnp.inf))
            min_sc[0] = jnp.minimum(min_sc[0], hm)

        hbuf[pl.ds(i * BC, BC)] = h.reshape(BC).astype(jnp.int32)

    @pl.when(i >= GRID_A)
    def _():
        j = i - GRID_A
        m = min_sc[0].astype(jnp.int32)
        o_ref[...] = hbuf[pl.ds(j * BS, BS)] - m


def kernel(inputs, a, b):
    xt = inputs.T
    a2 = a.reshape(1, D)
    out = pl.pallas_call(
        _body,
        grid=(GRID_A + GRID_B,),
        in_specs=[
            pl.BlockSpec((D, BC), lambda i: (0, jnp.minimum(i, GRID_A - 1))),
            pl.BlockSpec((1, D), lambda i: (0, 0)),
            pl.BlockSpec(memory_space=pltpu.SMEM),
        ],
        out_specs=pl.BlockSpec((BS,), lambda i: (jnp.maximum(i - GRID_A, 0),)),
        out_shape=jax.ShapeDtypeStruct((N_ROWS,), jnp.int32),
        scratch_shapes=[
            pltpu.VMEM((HBUF,), jnp.int32),
            pltpu.SMEM((1,), jnp.float32),
        ],
    )(xt, a2, b)
    return out


# speedup vs baseline: 1.0019x; 1.0011x over previous
import jax
import jax.numpy as jnp
from jax.experimental import pallas as pl
from jax.experimental.pallas import tpu as pltpu

BUCKET_W = 4.0
N_ROWS = 1_000_000
D = 64
BC = 32768
GRID_A = -(-N_ROWS // BC)
BS = 524288
GRID_B = -(-N_ROWS // BS)
HBUF = GRID_B * BS


def _body(x_ref, a_ref, b_ref, o_ref, hbuf, min_sc):
    i = pl.program_id(0)

    @pl.when(i < GRID_A)
    def _():
        proj = jax.lax.dot_general(
            a_ref[...], x_ref[...],
            dimension_numbers=(((1,), (0,)), ((), ())),
            preferred_element_type=jnp.float32,
        )
        h = jnp.floor((proj + b_ref[0]) * (1.0 / BUCKET_W))

        @pl.when(i == 0)
        def _():
            min_sc[0] = jnp.min(h)

        @pl.when((i > 0) & (i < GRID_A - 1))
        def _():
            min_sc[0] = jnp.minimum(min_sc[0], jnp.min(h))

        @pl.when(i == GRID_A - 1)
        def _():
            cols = i * BC + jax.lax.broadcasted_iota(jnp.int32, (1, BC), 1)
            hm = jnp.min(jnp.where(cols < N_ROWS, h, jnp.inf))
            min_sc[0] = jnp.minimum(min_sc[0], hm)

        hbuf[pl.ds(i * BC, BC)] = h.reshape(BC).astype(jnp.int32)

    @pl.when(i >= GRID_A)
    def _():
        j = i - GRID_A
        m = min_sc[0].astype(jnp.int32)
        o_ref[...] = hbuf[pl.ds(j * BS, BS)] - m


def kernel(inputs, a, b):
    xt = inputs.T
    a2 = a.reshape(1, D)
    out = pl.pallas_call(
        _body,
        grid=(GRID_A + GRID_B,),
        in_specs=[
            pl.BlockSpec((D, BC), lambda i: (0, jnp.minimum(i, GRID_A - 1))),
            pl.BlockSpec((1, D), lambda i: (0, 0)),
            pl.BlockSpec(memory_space=pltpu.SMEM),
        ],
        out_specs=pl.BlockSpec((BS,), lambda i: (jnp.maximum(i - GRID_A, 0),)),
        out_shape=jax.ShapeDtypeStruct((N_ROWS,), jnp.int32),
        scratch_shapes=[
            pltpu.VMEM((HBUF,), jnp.int32),
            pltpu.SMEM((1,), jnp.float32),
        ],
    )(xt, a2, b)
    return out
